# baseline (speedup 1.0000x reference)
import functools

import jax
import jax.numpy as jnp
import numpy as np
from jax import lax
from jax.experimental import pallas as pl
from jax.experimental.pallas import tpu as pltpu
from jax.experimental.pallas import tpu_sc as plsc

_NC = 2
_NS = 16
_NW = _NC * _NS
_LANES = 16
_CK = 128
_NB = 4


def _pe_const(length, dim):
  pos = np.arange(length, dtype=np.float32)[:, None]
  div = (1.0 / np.power(10000.0,
                        np.arange(0, dim * 2, 2, dtype=np.float32) / dim))
  pe = (pos * div[None, :]).astype(np.float32)
  pe[:, 0::2] = np.sin(pe[:, 0::2])
  pe[:, 1::2] = np.cos(pe[:, 1::2])
  return jnp.asarray(pe.reshape(length // 2, 2 * dim))


@functools.partial(jax.jit, static_argnames=("l_len", "batch", "dim"))
def _sc_lookup(idx, table2, pe, *, l_len, batch, dim):
  rows_total = l_len * batch
  per_w = rows_total // _NW
  nch = per_w // _CK
  ch_per_pos = batch // _CK
  main = (nch // _NB) * _NB
  wide = 2 * dim
  ck2 = _CK // 2

  mesh = plsc.VectorSubcoreMesh(core_axis_name="c", subcore_axis_name="s")

  @functools.partial(
      pl.kernel,
      out_type=jax.ShapeDtypeStruct((rows_total // 2, wide), jnp.float32),
      mesh=mesh,
      scratch_types=(
          [pltpu.VMEM((l_len // 2, wide), jnp.float32)]
          + [pltpu.VMEM((_CK,), jnp.int32) for _ in range(_NB)]
          + [pltpu.VMEM((ck2, wide), jnp.float32) for _ in range(_NB)]
          + [pltpu.SemaphoreType.DMA for _ in range(3 * _NB)]
      ),
  )
  def run(idx_hbm, table_hbm, pe_hbm, out_hbm, pe_v, *rest):
    idxv = rest[:_NB]
    bufs = rest[_NB:2 * _NB]
    isems = rest[2 * _NB:3 * _NB]
    gsems = rest[3 * _NB:4 * _NB]
    osems = rest[4 * _NB:]
    wid = lax.axis_index("s") * _NC + lax.axis_index("c")
    obase = wid * (per_w // 2)

    pltpu.sync_copy(pe_hbm, pe_v)

    def chunk_pos(cc):
      g = wid * nch + cc
      return g // ch_per_pos, (g % ch_per_pos) * _CK

    def idx_desc(cc, b):
      l, b0 = chunk_pos(cc)
      return pltpu.make_async_copy(idx_hbm.at[l, pl.ds(b0, _CK)], idxv[b],
                                   isems[b])

    def rows_start(b):
      for g in range(_CK // _LANES):
        v = idxv[b][pl.ds(g * _LANES, _LANES)]
        for j in range(_LANES):
          r = g * _LANES + j
          i = v[j]
          pltpu.async_copy(
              table_hbm.at[i >> 1, pl.ds((i & 1) * dim, dim)],
              bufs[b].at[r // 2, pl.ds((r % 2) * dim, dim)],
              gsems[b])

    def rows_drain(b):
      pltpu.make_async_copy(table_hbm.at[pl.ds(0, ck2)], bufs[b],
                            gsems[b]).wait()

    def out_desc(cc, b):
      return pltpu.make_async_copy(bufs[b],
                                   out_hbm.at[pl.ds(obase + cc * ck2, ck2)],
                                   osems[b])

    def pe_add(cc, b):
      l, _ = chunk_pos(cc)
      half = (l % 2) * dim
      pes = [pe_v[l // 2, pl.ds(half + k * _LANES, _LANES)]
             for k in range(dim // _LANES)]

      @plsc.parallel_loop(0, ck2, unroll=4)
      def _(r):
        for k in range(wide // _LANES):
          plsc.addupdate(bufs[b].at[r, pl.ds(k * _LANES, _LANES)],
                         pes[k % (dim // _LANES)])

    def step(cc, b, tail):
      idx_desc(cc, b).wait()
      rows_start(b)
      rows_drain(b)
      pe_add(cc, b)
      out_desc(cc, b).start()
      if tail:
        out_desc(cc - 1, (b - 1) % _NB).wait()
        if cc + (_NB - 1) < nch:
          idx_desc(cc + (_NB - 1), (b + _NB - 1) % _NB).start()
      else:
        @pl.when(cc >= 1)
        def _():
          out_desc(cc - 1, (b - 1) % _NB).wait()

        @pl.when(cc + (_NB - 1) < nch)
        def _():
          idx_desc(cc + (_NB - 1), (b + _NB - 1) % _NB).start()

    for b in range(_NB - 1):
      idx_desc(b, b).start()

    @pl.loop(0, main, step=_NB)
    def _(cc0):
      for b in range(_NB):
        step(cc0 + b, b, False)

    for cc in range(main, nch):
      step(cc, cc % _NB, True)
    out_desc(nch - 1, (nch - 1) % _NB).wait()

  return run(idx, table2, pe)


def kernel(input, table):
  l_len, batch, _ = input.shape
  vocab, dim = table.shape
  idx = input[:, :, 0]
  table2 = table.reshape(vocab // 2, 2 * dim)
  pe = _pe_const(l_len, dim)
  out = _sc_lookup(idx, table2, pe, l_len=l_len, batch=batch, dim=dim)
  return out.reshape(l_len, batch, dim)


# speedup vs baseline: 1.6432x; 1.6432x over previous
import functools

import jax
import jax.numpy as jnp
import numpy as np
from jax import lax
from jax.experimental import pallas as pl
from jax.experimental.pallas import tpu as pltpu
from jax.experimental.pallas import tpu_sc as plsc

_NC = 2
_NS = 16
_NW = _NC * _NS
_LANES = 16
_CK = 128
_NB = 4


def _pe_const(length, dim):
  pos = np.arange(length, dtype=np.float32)[:, None]
  div = (1.0 / np.power(10000.0,
                        np.arange(0, dim * 2, 2, dtype=np.float32) / dim))
  pe = (pos * div[None, :]).astype(np.float32)
  pe[:, 0::2] = np.sin(pe[:, 0::2])
  pe[:, 1::2] = np.cos(pe[:, 1::2])
  return jnp.asarray(pe.reshape(length // 2, 2 * dim))


@functools.partial(jax.jit, static_argnames=("l_len", "batch", "dim"))
def _sc_lookup(idx, table2, pe, *, l_len, batch, dim):
  rows_total = l_len * batch
  per_w = rows_total // _NW
  nch = per_w // _CK
  ch_per_pos = batch // _CK
  main = (nch // _NB) * _NB

  mesh = plsc.VectorSubcoreMesh(core_axis_name="c", subcore_axis_name="s")

  @functools.partial(
      pl.kernel,
      out_type=jax.ShapeDtypeStruct((l_len, batch, dim), jnp.float32),
      mesh=mesh,
      scratch_types=(
          [pltpu.VMEM((l_len // 2, 2 * dim), jnp.float32)]
          + [pltpu.VMEM((_CK,), jnp.int32) for _ in range(_NB)]
          + [pltpu.VMEM((_CK, dim), jnp.float32) for _ in range(_NB)]
          + [pltpu.SemaphoreType.DMA for _ in range(3 * _NB)]
      ),
      compiler_params=pltpu.CompilerParams(use_tc_tiling_on_sc=True),
  )
  def run(idx_hbm, table_hbm, pe_hbm, out_hbm, pe_v, *rest):
    idxv = rest[:_NB]
    bufs = rest[_NB:2 * _NB]
    isems = rest[2 * _NB:3 * _NB]
    gsems = rest[3 * _NB:4 * _NB]
    osems = rest[4 * _NB:]
    wid = lax.axis_index("s") * _NC + lax.axis_index("c")
    base = wid * per_w

    pltpu.sync_copy(pe_hbm, pe_v)

    def chunk_pos(cc):
      g = wid * nch + cc
      return g // ch_per_pos, (g % ch_per_pos) * _CK

    def idx_desc(cc, b):
      return pltpu.make_async_copy(idx_hbm.at[pl.ds(base + cc * _CK, _CK)],
                                   idxv[b], isems[b])

    def rows_start(b):
      @pl.loop(0, _CK // _LANES)
      def _(g):
        v = idxv[b][pl.ds(g * _LANES, _LANES)]
        for j in range(_LANES):
          pltpu.async_copy(table_hbm.at[v[j]],
                           bufs[b].at[g * _LANES + j], gsems[b])

    def rows_drain(b):
      pltpu.make_async_copy(table_hbm.at[pl.ds(0, dim)], bufs[b],
                            gsems[b]).wait()

    def out_desc(cc, b):
      l, b0 = chunk_pos(cc)
      return pltpu.make_async_copy(bufs[b], out_hbm.at[l, pl.ds(b0, _CK)],
                                   osems[b])

    def pe_add(cc, b):
      l, _ = chunk_pos(cc)
      half = (l % 2) * dim
      pes = [pe_v[l // 2, pl.ds(half + k * _LANES, _LANES)]
             for k in range(dim // _LANES)]

      @plsc.parallel_loop(0, _CK, unroll=8)
      def _(r):
        for k in range(dim // _LANES):
          plsc.addupdate(bufs[b].at[r, pl.ds(k * _LANES, _LANES)], pes[k])

    def step(cc, b, tail):
      rows_drain(b)
      pe_add(cc, b)
      out_desc(cc, b).start()
      if tail:
        out_desc(cc - 1, (b - 1) % _NB).wait()
        if cc + (_NB - 1) < nch:
          idx_desc(cc + (_NB - 1), (b + _NB - 1) % _NB).wait()
          rows_start((b + _NB - 1) % _NB)
        if cc + _NB < nch:
          idx_desc(cc + _NB, b).start()
      else:
        @pl.when(cc >= 1)
        def _():
          out_desc(cc - 1, (b - 1) % _NB).wait()

        @pl.when(cc + (_NB - 1) < nch)
        def _():
          idx_desc(cc + (_NB - 1), (b + _NB - 1) % _NB).wait()
          rows_start((b + _NB - 1) % _NB)

        @pl.when(cc + _NB < nch)
        def _():
          idx_desc(cc + _NB, b).start()

    for c in range(_NB):
      idx_desc(c, c).start()
    for c in range(_NB - 1):
      idx_desc(c, c).wait()
      rows_start(c)

    @pl.loop(0, main, step=_NB)
    def _(cc0):
      for b in range(_NB):
        step(cc0 + b, b, False)

    for cc in range(main, nch):
      step(cc, cc % _NB, True)
    out_desc(nch - 1, (nch - 1) % _NB).wait()

  return run(idx, table2, pe)


def kernel(input, table):
  l_len, batch, _ = input.shape
  vocab, dim = table.shape
  idx = input.reshape(l_len * batch)
  pe = _pe_const(l_len, dim)
  return _sc_lookup(idx, table, pe, l_len=l_len, batch=batch, dim=dim)
